# Initial kernel scaffold; baseline (speedup 1.0000x reference)
import jax
import jax.numpy as jnp
from jax.experimental import pallas as pl


def kernel(x, edge_index, W1, b1, W2, b2):
    raise NotImplementedError("write your pallas kernel here")


# speedup vs baseline: 17.4213x; 17.4213x over previous
import functools

import jax
import jax.numpy as jnp
from jax import lax
from jax.experimental import pallas as pl
from jax.experimental.pallas import tpu as pltpu
from jax.experimental.pallas import tpu_sc as plsc

N = 10000
E = 320000
K = 128
R = E // K
NC, NS = 2, 16
NW = NC * NS
N_DEG = 10240
DEG_SLICE = N_DEG // NS
N_PAD = 10240
ACC_SLICE = N_PAD // NS
BR = 1000


def _sc_mesh():
  return plsc.VectorSubcoreMesh(
      core_axis_name="c", subcore_axis_name="s", num_cores=NC, num_subcores=NS
  )


def _deg_partials(dst2d, zeros_deg):
  @functools.partial(
      pl.kernel,
      out_type=jax.ShapeDtypeStruct((NC, N_DEG), jnp.float32),
      mesh=_sc_mesh(),
      compiler_params=pltpu.CompilerParams(use_tc_tiling_on_sc=False),
      scratch_types=[
          pltpu.VMEM((K,), jnp.int32),
          pltpu.VMEM((K,), jnp.float32),
          pltpu.VMEM_SHARED((N_DEG,), jnp.float32),
      ],
  )
  def deg_kernel(dst_hbm, zero_hbm, out_hbm, idx_v, ones_v, acc):
    cid = lax.axis_index("c")
    sid = lax.axis_index("s")
    wid = sid * NC + cid
    for i in range(K // 16):
      ones_v[pl.ds(16 * i, 16)] = jnp.full((16,), 1.0, jnp.float32)
    pltpu.sync_copy(
        zero_hbm.at[pl.ds(sid * DEG_SLICE, DEG_SLICE)],
        acc.at[pl.ds(sid * DEG_SLICE, DEG_SLICE)],
    )
    plsc.subcore_barrier()
    nrows = R // NW + jnp.where(wid < (R % NW), 1, 0)

    def body(i, carry):
      row = wid + i * NW
      pltpu.sync_copy(dst_hbm.at[row], idx_v)
      pltpu.sync_copy(ones_v, acc.at[idx_v], add=True)
      return carry

    lax.fori_loop(0, nrows, body, 0)
    plsc.subcore_barrier()
    pltpu.sync_copy(
        acc.at[pl.ds(sid * DEG_SLICE, DEG_SLICE)],
        out_hbm.at[cid, pl.ds(sid * DEG_SLICE, DEG_SLICE)],
    )

  return deg_kernel(dst2d, zeros_deg)


def _aggregate(hs, src2d, dst2d, zeros_nd, d):
  @functools.partial(
      pl.kernel,
      out_type=jax.ShapeDtypeStruct((NC, N_PAD, d), jnp.float32),
      mesh=_sc_mesh(),
      compiler_params=pltpu.CompilerParams(use_tc_tiling_on_sc=False),
      scratch_types=[
          pltpu.VMEM((K,), jnp.int32),
          pltpu.VMEM((K,), jnp.int32),
          pltpu.VMEM((K, d), jnp.float32),
          pltpu.VMEM_SHARED((N_PAD, d), jnp.float32),
          pltpu.SemaphoreType.DMA,
      ],
  )
  def agg_kernel(hs_hbm, src_hbm, dst_hbm, zero_hbm, out_hbm, si_v, di_v, rows_v, acc, sem):
    cid = lax.axis_index("c")
    sid = lax.axis_index("s")
    wid = sid * NC + cid
    pltpu.sync_copy(
        zero_hbm.at[pl.ds(sid * ACC_SLICE, ACC_SLICE)],
        acc.at[pl.ds(sid * ACC_SLICE, ACC_SLICE)],
    )
    plsc.subcore_barrier()
    nrows = R // NW + jnp.where(wid < (R % NW), 1, 0)

    def body(i, carry):
      row = wid + i * NW
      pltpu.sync_copy(src_hbm.at[row], si_v)
      pltpu.sync_copy(dst_hbm.at[row], di_v)
      pltpu.async_copy(hs_hbm.at[si_v], rows_v, sem).wait()
      pltpu.sync_copy(rows_v, acc.at[di_v], add=True)
      return carry

    lax.fori_loop(0, nrows, body, 0)
    plsc.subcore_barrier()
    pltpu.sync_copy(
        acc.at[pl.ds(sid * ACC_SLICE, ACC_SLICE)],
        out_hbm.at[cid, pl.ds(sid * ACC_SLICE, ACC_SLICE)],
    )

  return agg_kernel(hs, src2d, dst2d, zeros_nd)


def _dense1(x, w1, degp):
  def body(x_ref, w_ref, d_ref, o_ref):
    dinv = lax.rsqrt(d_ref[:, 0] + d_ref[:, 1] + 1.0)
    o_ref[...] = (
        jnp.dot(x_ref[...], w_ref[...], preferred_element_type=jnp.float32)
        * dinv[:, None]
    )

  return pl.pallas_call(
      body,
      grid=(N // BR,),
      in_specs=[
          pl.BlockSpec((BR, 128), lambda i: (i, 0)),
          pl.BlockSpec((128, 64), lambda i: (0, 0)),
          pl.BlockSpec((BR, 2), lambda i: (i, 0)),
      ],
      out_specs=pl.BlockSpec((BR, 64), lambda i: (i, 0)),
      out_shape=jax.ShapeDtypeStruct((N, 64), jnp.float32),
  )(x, w1, degp)


def _dense2(p, hs1, degp, w2, b1):
  def body(p_ref, h_ref, d_ref, w_ref, b_ref, o_ref):
    dinv = lax.rsqrt(d_ref[:, 0] + d_ref[:, 1] + 1.0)
    o1 = (p_ref[0] + p_ref[1] + h_ref[...]) * dinv[:, None] + b_ref[...]
    o1 = jnp.maximum(o1, 0.0)
    o_ref[...] = (
        jnp.dot(o1, w_ref[...], preferred_element_type=jnp.float32)
        * dinv[:, None]
    )

  return pl.pallas_call(
      body,
      grid=(N // BR,),
      in_specs=[
          pl.BlockSpec((2, BR, 64), lambda i: (0, i, 0)),
          pl.BlockSpec((BR, 64), lambda i: (i, 0)),
          pl.BlockSpec((BR, 2), lambda i: (i, 0)),
          pl.BlockSpec((64, 128), lambda i: (0, 0)),
          pl.BlockSpec((1, 64), lambda i: (0, 0)),
      ],
      out_specs=pl.BlockSpec((BR, 128), lambda i: (i, 0)),
      out_shape=jax.ShapeDtypeStruct((N, 128), jnp.float32),
  )(p, hs1, degp, w2, b1)


def _dense3(q, hs2, degp, b2):
  def body(q_ref, h_ref, d_ref, b_ref, o_ref):
    dinv = lax.rsqrt(d_ref[:, 0] + d_ref[:, 1] + 1.0)
    o = (q_ref[0] + q_ref[1] + h_ref[...]) * dinv[:, None] + b_ref[...]
    o_ref[...] = jnp.maximum(o, 0.0)

  return pl.pallas_call(
      body,
      grid=(N // BR,),
      in_specs=[
          pl.BlockSpec((2, BR, 128), lambda i: (0, i, 0)),
          pl.BlockSpec((BR, 128), lambda i: (i, 0)),
          pl.BlockSpec((BR, 2), lambda i: (i, 0)),
          pl.BlockSpec((1, 128), lambda i: (0, 0)),
      ],
      out_specs=pl.BlockSpec((BR, 128), lambda i: (i, 0)),
      out_shape=jax.ShapeDtypeStruct((N, 128), jnp.float32),
  )(q, hs2, degp, b2)


def kernel(x, edge_index, W1, b1, W2, b2):
  ei = edge_index.astype(jnp.int32)
  src2d = ei[0].reshape(R, K)
  dst2d = ei[1].reshape(R, K)

  degp = _deg_partials(dst2d, jnp.zeros((N_DEG,), jnp.float32))[:, :N].T
  hs1 = _dense1(x, W1, degp)
  p = _aggregate(hs1, src2d, dst2d, jnp.zeros((N_PAD, 64), jnp.float32), 64)
  hs2 = _dense2(p, hs1, degp, W2, b1.reshape(1, 64))
  q = _aggregate(hs2, src2d, dst2d, jnp.zeros((N_PAD, 128), jnp.float32), 128)
  return _dense3(q, hs2, degp, b2.reshape(1, 128))


# speedup vs baseline: 33.8986x; 1.9458x over previous
import functools

import jax
import jax.numpy as jnp
from jax import lax
from jax.experimental import pallas as pl
from jax.experimental.pallas import tpu as pltpu
from jax.experimental.pallas import tpu_sc as plsc

N = 10000
E = 320000
K = 128
R = E // K
R_PAD = 2560
RPW = R_PAD // 32
NC, NS = 2, 16
NW = NC * NS
N_PAD = 10240
ACC_SLICE = N_PAD // NS
BR = 1000


def _sc_mesh():
  return plsc.VectorSubcoreMesh(
      core_axis_name="c", subcore_axis_name="s", num_cores=NC, num_subcores=NS
  )


def _deg_partials(dst2d, zeros_deg):
  nbuf = 10

  @functools.partial(
      pl.kernel,
      out_type=jax.ShapeDtypeStruct((NC, N_PAD), jnp.float32),
      mesh=_sc_mesh(),
      compiler_params=pltpu.CompilerParams(use_tc_tiling_on_sc=False),
      scratch_types=(
          [
              pltpu.VMEM((RPW, K), jnp.int32),
              pltpu.VMEM((K,), jnp.float32),
              pltpu.VMEM_SHARED((N_PAD,), jnp.float32),
          ]
          + [pltpu.SemaphoreType.DMA] * (nbuf + 2)
      ),
  )
  def deg_kernel(dst_hbm, zero_hbm, out_hbm, dslab, ones_v, acc, *sems):
    ssem = sems[:nbuf]
    isem, zsem = sems[nbuf], sems[nbuf + 1]
    cid = lax.axis_index("c")
    sid = lax.axis_index("s")
    wid = sid * NC + cid
    for i in range(K // 16):
      ones_v[pl.ds(16 * i, 16)] = jnp.full((16,), 1.0, jnp.float32)
    zdesc = pltpu.async_copy(
        zero_hbm.at[pl.ds(sid * ACC_SLICE, ACC_SLICE)],
        acc.at[pl.ds(sid * ACC_SLICE, ACC_SLICE)],
        zsem,
    )
    lo = jnp.minimum(wid * RPW, R - RPW)
    off = wid * RPW - lo
    idesc = pltpu.async_copy(dst_hbm.at[pl.ds(lo, RPW)], dslab, isem)
    zdesc.wait()
    idesc.wait()
    plsc.subcore_barrier()
    nrows = jnp.minimum(RPW, R - wid * RPW)
    ngroups = nrows // nbuf

    def body(g, carry):
      for b in range(nbuf):
        i = off + g * nbuf + b

        @pl.when(g > 0)
        def _wait():
          pltpu.make_async_copy(ones_v, acc.at[dslab.at[0]], ssem[b]).wait()

        pltpu.async_copy(ones_v, acc.at[dslab.at[i]], ssem[b], add=True)
      return carry

    lax.fori_loop(0, ngroups, body, 0)
    for b in range(nbuf):
      pltpu.make_async_copy(ones_v, acc.at[dslab.at[0]], ssem[b]).wait()
    plsc.subcore_barrier()
    pltpu.sync_copy(
        acc.at[pl.ds(sid * ACC_SLICE, ACC_SLICE)],
        out_hbm.at[cid, pl.ds(sid * ACC_SLICE, ACC_SLICE)],
    )

  return deg_kernel(dst2d, zeros_deg)


def _aggregate(hs, src2d, dst2d, zeros_nd, d, nbuf):
  @functools.partial(
      pl.kernel,
      out_type=jax.ShapeDtypeStruct((NC, N_PAD, d), jnp.float32),
      mesh=_sc_mesh(),
      compiler_params=pltpu.CompilerParams(use_tc_tiling_on_sc=False),
      scratch_types=(
          [pltpu.VMEM((RPW, K), jnp.int32), pltpu.VMEM((RPW, K), jnp.int32)]
          + [pltpu.VMEM((K, d), jnp.float32) for _ in range(nbuf)]
          + [pltpu.VMEM_SHARED((N_PAD, d), jnp.float32)]
          + [pltpu.SemaphoreType.DMA] * (2 * nbuf + 3)
      ),
  )
  def agg_kernel(hs_hbm, src_hbm, dst_hbm, zero_hbm, out_hbm, *scr):
    sslab, dslab = scr[0], scr[1]
    rows = scr[2 : 2 + nbuf]
    acc = scr[2 + nbuf]
    sems = scr[3 + nbuf :]
    gsem = sems[:nbuf]
    ssem = sems[nbuf : 2 * nbuf]
    isem, jsem, zsem = sems[2 * nbuf], sems[2 * nbuf + 1], sems[2 * nbuf + 2]
    cid = lax.axis_index("c")
    sid = lax.axis_index("s")
    wid = sid * NC + cid
    zdesc = pltpu.async_copy(
        zero_hbm.at[pl.ds(sid * ACC_SLICE, ACC_SLICE)],
        acc.at[pl.ds(sid * ACC_SLICE, ACC_SLICE)],
        zsem,
    )
    lo = jnp.minimum(wid * RPW, R - RPW)
    off = wid * RPW - lo
    idesc = pltpu.async_copy(src_hbm.at[pl.ds(lo, RPW)], sslab, isem)
    jdesc = pltpu.async_copy(dst_hbm.at[pl.ds(lo, RPW)], dslab, jsem)
    zdesc.wait()
    idesc.wait()
    jdesc.wait()
    plsc.subcore_barrier()
    nrows = jnp.minimum(RPW, R - wid * RPW)
    ngroups = nrows // nbuf

    def body(g, carry):
      gd = []
      for b in range(nbuf):
        i = off + g * nbuf + b

        @pl.when(g > 0)
        def _wait():
          pltpu.make_async_copy(rows[b], acc.at[dslab.at[0]], ssem[b]).wait()

        gd.append(pltpu.async_copy(hs_hbm.at[sslab.at[i]], rows[b], gsem[b]))
      for b in range(nbuf):
        i = off + g * nbuf + b
        gd[b].wait()
        pltpu.async_copy(rows[b], acc.at[dslab.at[i]], ssem[b], add=True)
      return carry

    lax.fori_loop(0, ngroups, body, 0)
    for b in range(nbuf):
      pltpu.make_async_copy(rows[b], acc.at[dslab.at[0]], ssem[b]).wait()
    plsc.subcore_barrier()
    pltpu.sync_copy(
        acc.at[pl.ds(sid * ACC_SLICE, ACC_SLICE)],
        out_hbm.at[cid, pl.ds(sid * ACC_SLICE, ACC_SLICE)],
    )

  return agg_kernel(hs, src2d, dst2d, zeros_nd)


def _dense1(x, w1, degp):
  def body(x_ref, w_ref, d_ref, o_ref):
    dinv = lax.rsqrt(d_ref[:, 0] + d_ref[:, 1] + 1.0)
    o_ref[...] = (
        jnp.dot(x_ref[...], w_ref[...], preferred_element_type=jnp.float32)
        * dinv[:, None]
    )

  return pl.pallas_call(
      body,
      grid=(N // BR,),
      in_specs=[
          pl.BlockSpec((BR, 128), lambda i: (i, 0)),
          pl.BlockSpec((128, 64), lambda i: (0, 0)),
          pl.BlockSpec((BR, 2), lambda i: (i, 0)),
      ],
      out_specs=pl.BlockSpec((BR, 64), lambda i: (i, 0)),
      out_shape=jax.ShapeDtypeStruct((N, 64), jnp.float32),
  )(x, w1, degp)


def _dense2(p, hs1, degp, w2, b1):
  def body(p_ref, h_ref, d_ref, w_ref, b_ref, oa_ref, ob_ref):
    dinv = lax.rsqrt(d_ref[:, 0] + d_ref[:, 1] + 1.0)
    o1 = (p_ref[0] + p_ref[1] + h_ref[...]) * dinv[:, None] + b_ref[...]
    o1 = jnp.maximum(o1, 0.0)
    h2 = jnp.dot(o1, w_ref[...], preferred_element_type=jnp.float32)
    oa_ref[...] = h2[:, :64] * dinv[:, None]
    ob_ref[...] = h2[:, 64:] * dinv[:, None]

  return pl.pallas_call(
      body,
      grid=(N // BR,),
      in_specs=[
          pl.BlockSpec((2, BR, 64), lambda i: (0, i, 0)),
          pl.BlockSpec((BR, 64), lambda i: (i, 0)),
          pl.BlockSpec((BR, 2), lambda i: (i, 0)),
          pl.BlockSpec((64, 128), lambda i: (0, 0)),
          pl.BlockSpec((1, 64), lambda i: (0, 0)),
      ],
      out_specs=[
          pl.BlockSpec((BR, 64), lambda i: (i, 0)),
          pl.BlockSpec((BR, 64), lambda i: (i, 0)),
      ],
      out_shape=[
          jax.ShapeDtypeStruct((N, 64), jnp.float32),
          jax.ShapeDtypeStruct((N, 64), jnp.float32),
      ],
  )(p, hs1, degp, w2, b1)


def _dense3(qa, qb, hs2a, hs2b, degp, b2):
  def body(qa_ref, qb_ref, ha_ref, hb_ref, d_ref, b_ref, o_ref):
    dinv = lax.rsqrt(d_ref[:, 0] + d_ref[:, 1] + 1.0)
    oa = (qa_ref[0] + qa_ref[1] + ha_ref[...]) * dinv[:, None] + b_ref[:, :64]
    ob = (qb_ref[0] + qb_ref[1] + hb_ref[...]) * dinv[:, None] + b_ref[:, 64:]
    o_ref[:, :64] = jnp.maximum(oa, 0.0)
    o_ref[:, 64:] = jnp.maximum(ob, 0.0)

  return pl.pallas_call(
      body,
      grid=(N // BR,),
      in_specs=[
          pl.BlockSpec((2, BR, 64), lambda i: (0, i, 0)),
          pl.BlockSpec((2, BR, 64), lambda i: (0, i, 0)),
          pl.BlockSpec((BR, 64), lambda i: (i, 0)),
          pl.BlockSpec((BR, 64), lambda i: (i, 0)),
          pl.BlockSpec((BR, 2), lambda i: (i, 0)),
          pl.BlockSpec((1, 128), lambda i: (0, 0)),
      ],
      out_specs=pl.BlockSpec((BR, 128), lambda i: (i, 0)),
      out_shape=jax.ShapeDtypeStruct((N, 128), jnp.float32),
  )(qa, qb, hs2a, hs2b, degp, b2)


def kernel(x, edge_index, W1, b1, W2, b2):
  ei = edge_index.astype(jnp.int32)
  src2d = ei[0].reshape(R, K)
  dst2d = ei[1].reshape(R, K)
  zeros64 = jnp.zeros((N_PAD, 64), jnp.float32)

  degp = _deg_partials(dst2d, jnp.zeros((N_PAD,), jnp.float32))[:, :N].T
  hs1 = _dense1(x, W1, degp)
  p = _aggregate(hs1, src2d, dst2d, zeros64, 64, 5)
  hs2a, hs2b = _dense2(p, hs1, degp, W2, b1.reshape(1, 64))
  qa = _aggregate(hs2a, src2d, dst2d, zeros64, 64, 5)
  qb = _aggregate(hs2b, src2d, dst2d, zeros64, 64, 5)
  return _dense3(qa, qb, hs2a, hs2b, degp, b2.reshape(1, 128))


# speedup vs baseline: 34.8868x; 1.0292x over previous
import functools

import jax
import jax.numpy as jnp
from jax import lax
from jax.experimental import pallas as pl
from jax.experimental.pallas import tpu as pltpu
from jax.experimental.pallas import tpu_sc as plsc

N = 10000
E = 320000
K = 128
R = E // K
R_PAD = 2560
RPW = R_PAD // 32
NC, NS = 2, 16
NW = NC * NS
N_PAD = 10240
ACC_SLICE = N_PAD // NS
BR = 1000


def _sc_mesh():
  return plsc.VectorSubcoreMesh(
      core_axis_name="c", subcore_axis_name="s", num_cores=NC, num_subcores=NS
  )


def _deg_partials(dst2d, zeros_deg):
  nbuf = 10

  @functools.partial(
      pl.kernel,
      out_type=jax.ShapeDtypeStruct((NC, N_PAD), jnp.float32),
      mesh=_sc_mesh(),
      compiler_params=pltpu.CompilerParams(use_tc_tiling_on_sc=False),
      scratch_types=(
          [
              pltpu.VMEM((RPW, K), jnp.int32),
              pltpu.VMEM((K,), jnp.float32),
              pltpu.VMEM_SHARED((N_PAD,), jnp.float32),
          ]
          + [pltpu.SemaphoreType.DMA] * (nbuf + 2)
      ),
  )
  def deg_kernel(dst_hbm, zero_hbm, out_hbm, dslab, ones_v, acc, *sems):
    ssem = sems[:nbuf]
    isem, zsem = sems[nbuf], sems[nbuf + 1]
    cid = lax.axis_index("c")
    sid = lax.axis_index("s")
    wid = sid * NC + cid
    for i in range(K // 16):
      ones_v[pl.ds(16 * i, 16)] = jnp.full((16,), 1.0, jnp.float32)
    zdesc = pltpu.async_copy(
        zero_hbm.at[pl.ds(sid * ACC_SLICE, ACC_SLICE)],
        acc.at[pl.ds(sid * ACC_SLICE, ACC_SLICE)],
        zsem,
    )
    lo = jnp.minimum(wid * RPW, R - RPW)
    off = wid * RPW - lo
    idesc = pltpu.async_copy(dst_hbm.at[pl.ds(lo, RPW)], dslab, isem)
    zdesc.wait()
    idesc.wait()
    plsc.subcore_barrier()
    nrows = jnp.minimum(RPW, R - wid * RPW)
    ngroups = nrows // nbuf

    def body(g, carry):
      for b in range(nbuf):
        i = off + g * nbuf + b

        @pl.when(g > 0)
        def _wait():
          pltpu.make_async_copy(ones_v, acc.at[dslab.at[0]], ssem[b]).wait()

        pltpu.async_copy(ones_v, acc.at[dslab.at[i]], ssem[b], add=True)
      return carry

    lax.fori_loop(0, ngroups, body, 0)
    for b in range(nbuf):
      pltpu.make_async_copy(ones_v, acc.at[dslab.at[0]], ssem[b]).wait()
    plsc.subcore_barrier()
    pltpu.sync_copy(
        acc.at[pl.ds(sid * ACC_SLICE, ACC_SLICE)],
        out_hbm.at[cid, pl.ds(sid * ACC_SLICE, ACC_SLICE)],
    )

  return deg_kernel(dst2d, zeros_deg)


def _aggregate(hs, src2d, dst2d, zeros_nd, d, nbuf):
  @functools.partial(
      pl.kernel,
      out_type=jax.ShapeDtypeStruct((NC, N_PAD, d), jnp.float32),
      mesh=_sc_mesh(),
      compiler_params=pltpu.CompilerParams(use_tc_tiling_on_sc=False),
      scratch_types=(
          [pltpu.VMEM((RPW, K), jnp.int32), pltpu.VMEM((RPW, K), jnp.int32)]
          + [pltpu.VMEM((K, d), jnp.float32) for _ in range(nbuf)]
          + [pltpu.VMEM_SHARED((N_PAD, d), jnp.float32)]
          + [pltpu.SemaphoreType.DMA] * (2 * nbuf + 3)
      ),
  )
  def agg_kernel(hs_hbm, src_hbm, dst_hbm, zero_hbm, out_hbm, *scr):
    sslab, dslab = scr[0], scr[1]
    rows = scr[2 : 2 + nbuf]
    acc = scr[2 + nbuf]
    sems = scr[3 + nbuf :]
    gsem = sems[:nbuf]
    ssem = sems[nbuf : 2 * nbuf]
    isem, jsem, zsem = sems[2 * nbuf], sems[2 * nbuf + 1], sems[2 * nbuf + 2]
    cid = lax.axis_index("c")
    sid = lax.axis_index("s")
    wid = sid * NC + cid
    zdesc = pltpu.async_copy(
        zero_hbm.at[pl.ds(sid * ACC_SLICE, ACC_SLICE)],
        acc.at[pl.ds(sid * ACC_SLICE, ACC_SLICE)],
        zsem,
    )
    lo = jnp.minimum(wid * RPW, R - RPW)
    off = wid * RPW - lo
    idesc = pltpu.async_copy(src_hbm.at[pl.ds(lo, RPW)], sslab, isem)
    jdesc = pltpu.async_copy(dst_hbm.at[pl.ds(lo, RPW)], dslab, jsem)
    zdesc.wait()
    idesc.wait()
    jdesc.wait()
    plsc.subcore_barrier()
    nrows = jnp.minimum(RPW, R - wid * RPW)
    ngroups = nrows // nbuf

    def body(g, carry):
      gd = []
      for b in range(nbuf):
        i = off + g * nbuf + b

        @pl.when(g > 0)
        def _wait():
          pltpu.make_async_copy(rows[b], acc.at[dslab.at[0]], ssem[b]).wait()

        gd.append(pltpu.async_copy(hs_hbm.at[sslab.at[i]], rows[b], gsem[b]))
      for b in range(nbuf):
        i = off + g * nbuf + b
        gd[b].wait()
        pltpu.async_copy(rows[b], acc.at[dslab.at[i]], ssem[b], add=True)
      return carry

    lax.fori_loop(0, ngroups, body, 0)
    for b in range(nbuf):
      pltpu.make_async_copy(rows[b], acc.at[dslab.at[0]], ssem[b]).wait()

    def tail(t, carry):
      i = off + ngroups * nbuf + t
      pltpu.async_copy(hs_hbm.at[sslab.at[i]], rows[0], gsem[0]).wait()
      pltpu.async_copy(rows[0], acc.at[dslab.at[i]], ssem[0], add=True)
      pltpu.make_async_copy(rows[0], acc.at[dslab.at[0]], ssem[0]).wait()
      return carry

    lax.fori_loop(0, nrows - ngroups * nbuf, tail, 0)
    plsc.subcore_barrier()
    pltpu.sync_copy(
        acc.at[pl.ds(sid * ACC_SLICE, ACC_SLICE)],
        out_hbm.at[cid, pl.ds(sid * ACC_SLICE, ACC_SLICE)],
    )

  return agg_kernel(hs, src2d, dst2d, zeros_nd)


def _dense1(x, w1, degp):
  def body(x_ref, w_ref, d_ref, o_ref):
    dinv = lax.rsqrt(d_ref[:, 0] + d_ref[:, 1] + 1.0)
    o_ref[...] = (
        jnp.dot(x_ref[...], w_ref[...], preferred_element_type=jnp.float32)
        * dinv[:, None]
    )

  return pl.pallas_call(
      body,
      grid=(N // BR,),
      in_specs=[
          pl.BlockSpec((BR, 128), lambda i: (i, 0)),
          pl.BlockSpec((128, 64), lambda i: (0, 0)),
          pl.BlockSpec((BR, 2), lambda i: (i, 0)),
      ],
      out_specs=pl.BlockSpec((BR, 64), lambda i: (i, 0)),
      out_shape=jax.ShapeDtypeStruct((N, 64), jnp.float32),
  )(x, w1, degp)


def _dense2(p, hs1, degp, w2, b1):
  def body(p_ref, h_ref, d_ref, w_ref, b_ref, oa_ref, ob_ref):
    dinv = lax.rsqrt(d_ref[:, 0] + d_ref[:, 1] + 1.0)
    o1 = (p_ref[0] + p_ref[1] + h_ref[...]) * dinv[:, None] + b_ref[...]
    o1 = jnp.maximum(o1, 0.0)
    h2 = jnp.dot(o1, w_ref[...], preferred_element_type=jnp.float32)
    oa_ref[...] = h2[:, :64] * dinv[:, None]
    ob_ref[...] = h2[:, 64:] * dinv[:, None]

  return pl.pallas_call(
      body,
      grid=(N // BR,),
      in_specs=[
          pl.BlockSpec((2, BR, 64), lambda i: (0, i, 0)),
          pl.BlockSpec((BR, 64), lambda i: (i, 0)),
          pl.BlockSpec((BR, 2), lambda i: (i, 0)),
          pl.BlockSpec((64, 128), lambda i: (0, 0)),
          pl.BlockSpec((1, 64), lambda i: (0, 0)),
      ],
      out_specs=[
          pl.BlockSpec((BR, 64), lambda i: (i, 0)),
          pl.BlockSpec((BR, 64), lambda i: (i, 0)),
      ],
      out_shape=[
          jax.ShapeDtypeStruct((N, 64), jnp.float32),
          jax.ShapeDtypeStruct((N, 64), jnp.float32),
      ],
  )(p, hs1, degp, w2, b1)


def _dense3(qa, qb, hs2a, hs2b, degp, b2):
  def body(qa_ref, qb_ref, ha_ref, hb_ref, d_ref, b_ref, o_ref):
    dinv = lax.rsqrt(d_ref[:, 0] + d_ref[:, 1] + 1.0)
    oa = (qa_ref[0] + qa_ref[1] + ha_ref[...]) * dinv[:, None] + b_ref[:, :64]
    ob = (qb_ref[0] + qb_ref[1] + hb_ref[...]) * dinv[:, None] + b_ref[:, 64:]
    o_ref[:, :64] = jnp.maximum(oa, 0.0)
    o_ref[:, 64:] = jnp.maximum(ob, 0.0)

  return pl.pallas_call(
      body,
      grid=(N // BR,),
      in_specs=[
          pl.BlockSpec((2, BR, 64), lambda i: (0, i, 0)),
          pl.BlockSpec((2, BR, 64), lambda i: (0, i, 0)),
          pl.BlockSpec((BR, 64), lambda i: (i, 0)),
          pl.BlockSpec((BR, 64), lambda i: (i, 0)),
          pl.BlockSpec((BR, 2), lambda i: (i, 0)),
          pl.BlockSpec((1, 128), lambda i: (0, 0)),
      ],
      out_specs=pl.BlockSpec((BR, 128), lambda i: (i, 0)),
      out_shape=jax.ShapeDtypeStruct((N, 128), jnp.float32),
  )(qa, qb, hs2a, hs2b, degp, b2)


def kernel(x, edge_index, W1, b1, W2, b2):
  ei = edge_index.astype(jnp.int32)
  src2d = ei[0].reshape(R, K)
  dst2d = ei[1].reshape(R, K)
  zeros64 = jnp.zeros((N_PAD, 64), jnp.float32)

  degp = _deg_partials(dst2d, jnp.zeros((N_PAD,), jnp.float32))[:, :N].T
  hs1 = _dense1(x, W1, degp)
  p = _aggregate(hs1, src2d, dst2d, zeros64, 64, 8)
  hs2a, hs2b = _dense2(p, hs1, degp, W2, b1.reshape(1, 64))
  qa = _aggregate(hs2a, src2d, dst2d, zeros64, 64, 8)
  qb = _aggregate(hs2b, src2d, dst2d, zeros64, 64, 8)
  return _dense3(qa, qb, hs2a, hs2b, degp, b2.reshape(1, 128))
